# baseline (speedup 1.0000x reference)
import functools

import jax
import jax.numpy as jnp
from jax import lax
from jax.experimental import pallas as pl
from jax.experimental.pallas import tpu as pltpu
from jax.experimental.pallas import tpu_sc as plsc

_L = 16


def _make_sc_gather_score(B, D, NNEG):
    NW = 32
    BW = B // NW
    CB = 64
    NCH = BW // CB
    KD = D // _L
    NG = CB // _L

    mesh = plsc.VectorSubcoreMesh(
        core_axis_name="c", subcore_axis_name="s", num_cores=2, num_subcores=16
    )

    @functools.partial(
        pl.kernel,
        out_type=jax.ShapeDtypeStruct((B, 2 * _L), jnp.float32),
        mesh=mesh,
        scratch_types=[
            pltpu.VMEM((CB,), jnp.int32),
            pltpu.VMEM((CB,), jnp.int32),
            pltpu.VMEM((CB * NNEG,), jnp.int32),
            pltpu.VMEM((2, CB * D), jnp.float32),
            pltpu.VMEM((2, CB * D), jnp.float32),
            pltpu.VMEM((2, NNEG * CB * D), jnp.float32),
            pltpu.VMEM((CB, 2 * _L), jnp.float32),
            pltpu.SemaphoreType.DMA,
            pltpu.SemaphoreType.DMA,
        ],
    )
    def sc_fn(u_pos_h, v_pos_h, vneg_h, u_tab, v_tab, out_h,
              u_idx, v_idx, n_idx, u_rows, v_rows, n_rows, pbuf, sem0, sem1):
        wid = lax.axis_index("c") * 16 + lax.axis_index("s")
        base = wid * BW

        sems = (sem0, sem1)

        def fire(c, p):
            s = sems[p]
            pltpu.sync_copy(u_pos_h.at[pl.ds(base + c * CB, CB)], u_idx)
            pltpu.sync_copy(v_pos_h.at[pl.ds(base + c * CB, CB)], v_idx)
            pltpu.sync_copy(
                vneg_h.at[pl.ds((base + c * CB) * NNEG, CB * NNEG)], n_idx)

            def issue(g, carry):
                uvec = u_idx[pl.ds(g * _L, _L)] * D
                vvec = v_idx[pl.ds(g * _L, _L)] * D
                nvec = [n_idx[pl.ds(g * _L * NNEG + q * _L, _L)] * D
                        for q in range(NNEG)]
                for t in range(_L):
                    i = g * _L + t
                    di = pl.multiple_of(i * D, D)
                    pltpu.async_copy(
                        u_tab.at[pl.ds(pl.multiple_of(uvec[t], D), D)],
                        u_rows.at[p, pl.ds(di, D)], s)
                    pltpu.async_copy(
                        v_tab.at[pl.ds(pl.multiple_of(vvec[t], D), D)],
                        v_rows.at[p, pl.ds(di, D)], s)
                    for j in range(NNEG):
                        q, r = divmod(t * NNEG + j, _L)
                        pltpu.async_copy(
                            v_tab.at[pl.ds(pl.multiple_of(nvec[q][r], D), D)],
                            n_rows.at[p, pl.ds(pl.multiple_of(
                                (j * CB + i) * D, D), D)], s)
                return carry

            lax.fori_loop(0, NG, issue, 0)

        def drain(p):
            s = sems[p]
            pltpu.make_async_copy(
                u_tab.at[pl.ds(0, CB * D)], u_rows.at[p], s).wait()
            pltpu.make_async_copy(
                u_tab.at[pl.ds(0, CB * D)], v_rows.at[p], s).wait()
            pltpu.make_async_copy(
                u_tab.at[pl.ds(0, NNEG * CB * D)], n_rows.at[p], s).wait()

        def compute_chunk(p):
            ub, vb, nb, pb = u_rows.at[p], v_rows.at[p], n_rows.at[p], pbuf

            def body(i, carry):
                pos = None
                neg = None
                for k in range(KD):
                    o = k * _L
                    io = pl.multiple_of(i * D + o, _L)
                    u = ub[pl.ds(io, _L)]
                    pp = u * vb[pl.ds(io, _L)]
                    ns = nb[pl.ds(io, _L)]
                    for j in range(1, NNEG):
                        ns = ns + nb[pl.ds(pl.multiple_of(
                            (j * CB + i) * D + o, _L), _L)]
                    nn = u * ns
                    pos = pp if pos is None else pos + pp
                    neg = nn if neg is None else neg + nn
                pb[i, pl.ds(0, _L)] = pos
                pb[i, pl.ds(_L, _L)] = neg
                return carry

            lax.fori_loop(0, CB, body, 0, unroll=2)

        def emit(c, p):
            compute_chunk(p)
            pltpu.sync_copy(pbuf, out_h.at[pl.ds(base + c * CB, CB)])

        fire(0, 0)

        def pair(k, carry):
            c0 = 2 * k
            fire(c0 + 1, 1)
            drain(0)
            emit(c0, 0)

            @pl.when(k + 1 < NCH // 2)
            def _():
                fire(c0 + 2, 0)

            drain(1)
            emit(c0 + 1, 1)
            return carry

        lax.fori_loop(0, NCH // 2, pair, 0)

    return sc_fn


def _finish(part, B):
    def body(x_ref, o_ref):
        x = x_ref[...]
        pos = jnp.sum(x[:, :_L], axis=1)
        neg = jnp.sum(x[:, _L:], axis=1)
        tot = jax.nn.log_sigmoid(pos) + jax.nn.log_sigmoid(-neg)
        o_ref[0, 0] = -jnp.sum(tot) / B

    return pl.pallas_call(
        body,
        out_shape=jax.ShapeDtypeStruct((1, 1), jnp.float32),
        in_specs=[pl.BlockSpec(memory_space=pltpu.VMEM)],
        out_specs=pl.BlockSpec(memory_space=pltpu.SMEM),
    )(part)


def kernel(u_pos, v_pos, v_neg, batch_size, U, V):
    B = u_pos.shape[0]
    D = U.shape[1]
    NNEG = v_neg.shape[1]
    vneg_flat = v_neg.reshape(B * NNEG)
    sc_fn = _make_sc_gather_score(B, D, NNEG)
    part = sc_fn(u_pos, v_pos, vneg_flat, U.reshape(-1), V.reshape(-1))
    out = _finish(part, B)
    return out[0, 0]


# speedup vs baseline: 1.5203x; 1.5203x over previous
import functools

import jax
import jax.numpy as jnp
from jax import lax
from jax.experimental import pallas as pl
from jax.experimental.pallas import tpu as pltpu
from jax.experimental.pallas import tpu_sc as plsc

_L = 16


def _make_sc_gather_score(B, D, NNEG):
    NW = 32
    BW = B // NW
    CB = 64
    NCH = BW // CB
    KD = D // _L
    NG = CB // _L

    mesh = plsc.VectorSubcoreMesh(
        core_axis_name="c", subcore_axis_name="s", num_cores=2, num_subcores=16
    )

    @functools.partial(
        pl.kernel,
        out_type=jax.ShapeDtypeStruct((B, 2 * _L), jnp.float32),
        mesh=mesh,
        scratch_types=[
            pltpu.VMEM((CB,), jnp.int32),
            pltpu.VMEM((CB,), jnp.int32),
            pltpu.VMEM((CB * NNEG,), jnp.int32),
            pltpu.VMEM((2, CB, D), jnp.float32),
            pltpu.VMEM((2, CB, D), jnp.float32),
            pltpu.VMEM((2, NNEG, CB, D), jnp.float32),
            pltpu.VMEM((CB, 2 * _L), jnp.float32),
            pltpu.SemaphoreType.DMA,
            pltpu.SemaphoreType.DMA,
        ],
        compiler_params=pltpu.CompilerParams(needs_layout_passes=False),
    )
    def sc_fn(u_pos_h, v_pos_h, vneg_h, u_tab, v_tab, out_h,
              u_idx, v_idx, n_idx, u_rows, v_rows, n_rows, pbuf, sem0, sem1):
        wid = lax.axis_index("c") * 16 + lax.axis_index("s")
        base = wid * BW

        sems = (sem0, sem1)

        def fire(c, p):
            s = sems[p]
            pltpu.sync_copy(u_pos_h.at[pl.ds(base + c * CB, CB)], u_idx)
            pltpu.sync_copy(v_pos_h.at[pl.ds(base + c * CB, CB)], v_idx)
            pltpu.sync_copy(
                vneg_h.at[pl.ds((base + c * CB) * NNEG, CB * NNEG)], n_idx)

            def issue(g, carry):
                uvec = u_idx[pl.ds(g * _L, _L)]
                vvec = v_idx[pl.ds(g * _L, _L)]
                nvec = [n_idx[pl.ds(g * _L * NNEG + q * _L, _L)]
                        for q in range(NNEG)]
                for t in range(_L):
                    i = g * _L + t
                    pltpu.async_copy(u_tab.at[uvec[t]], u_rows.at[p, i], s)
                    pltpu.async_copy(v_tab.at[vvec[t]], v_rows.at[p, i], s)
                    for j in range(NNEG):
                        q, r = divmod(t * NNEG + j, _L)
                        pltpu.async_copy(
                            v_tab.at[nvec[q][r]], n_rows.at[p, j, i], s)
                return carry

            lax.fori_loop(0, NG, issue, 0)

        def drain(p):
            s = sems[p]
            pltpu.make_async_copy(
                u_tab.at[pl.ds(0, CB)], u_rows.at[p], s).wait()
            pltpu.make_async_copy(
                u_tab.at[pl.ds(0, CB)], v_rows.at[p], s).wait()
            for j in range(NNEG):
                pltpu.make_async_copy(
                    u_tab.at[pl.ds(0, CB)], n_rows.at[p, j], s).wait()

        def compute_chunk(p):
            ub, vb, nb, pb = (u_rows.at[p], v_rows.at[p], n_rows.at[p],
                              pbuf)

            def body(i, carry):
                pos = None
                neg = None
                for k in range(KD):
                    sl = pl.ds(k * _L, _L)
                    u = ub[i, sl]
                    pp = u * vb[i, sl]
                    ns = nb[0, i, sl]
                    for j in range(1, NNEG):
                        ns = ns + nb[j, i, sl]
                    nn = u * ns
                    pos = pp if pos is None else pos + pp
                    neg = nn if neg is None else neg + nn
                pb[i, pl.ds(0, _L)] = pos
                pb[i, pl.ds(_L, _L)] = neg
                return carry

            lax.fori_loop(0, CB, body, 0, unroll=2)

        def emit(c, p):
            compute_chunk(p)
            pltpu.sync_copy(pbuf, out_h.at[pl.ds(base + c * CB, CB)])

        fire(0, 0)

        def pair(k, carry):
            c0 = 2 * k
            fire(c0 + 1, 1)
            drain(0)
            emit(c0, 0)

            @pl.when(k + 1 < NCH // 2)
            def _():
                fire(c0 + 2, 0)

            drain(1)
            emit(c0 + 1, 1)
            return carry

        lax.fori_loop(0, NCH // 2, pair, 0)

    return sc_fn


def _finish(part, B):
    def body(x_ref, o_ref):
        x = x_ref[...]
        pos = jnp.sum(x[:, :_L], axis=1)
        neg = jnp.sum(x[:, _L:], axis=1)
        tot = jax.nn.log_sigmoid(pos) + jax.nn.log_sigmoid(-neg)
        o_ref[0, 0] = -jnp.sum(tot) / B

    return pl.pallas_call(
        body,
        out_shape=jax.ShapeDtypeStruct((1, 1), jnp.float32),
        in_specs=[pl.BlockSpec(memory_space=pltpu.VMEM)],
        out_specs=pl.BlockSpec(memory_space=pltpu.SMEM),
    )(part)


def kernel(u_pos, v_pos, v_neg, batch_size, U, V):
    B = u_pos.shape[0]
    D = U.shape[1]
    NNEG = v_neg.shape[1]
    vneg_flat = v_neg.reshape(B * NNEG)
    sc_fn = _make_sc_gather_score(B, D, NNEG)
    part = sc_fn(u_pos, v_pos, vneg_flat, U, V)
    out = _finish(part, B)
    return out[0, 0]


# speedup vs baseline: 1.8070x; 1.1886x over previous
import functools

import jax
import jax.numpy as jnp
from jax import lax
from jax.experimental import pallas as pl
from jax.experimental.pallas import tpu as pltpu
from jax.experimental.pallas import tpu_sc as plsc

_L = 16


_BKV = 8192


def _pack_table(xt):
    D, V = xt.shape
    grid = -(-V // _BKV)

    def body(x_ref, o_ref):
        y = x_ref[...].T
        o_ref[...] = jnp.concatenate([y[:_BKV // 2], y[_BKV // 2:]], axis=1)

    return pl.pallas_call(
        body,
        grid=(grid,),
        in_specs=[pl.BlockSpec((D, _BKV), lambda i: (0, i))],
        out_specs=pl.BlockSpec((_BKV // 2, 2 * D), lambda i: (i, 0)),
        out_shape=jax.ShapeDtypeStruct((grid * _BKV // 2, 2 * D), jnp.float32),
    )(xt)


def _wide_row(x):
    return ((x >> 13) << 12) + (x & 4095)


def _half(x):
    return (x >> 12) & 1


def _make_sc_gather_score(B, D, NNEG):
    NW = 32
    BW = B // NW
    CB = 16
    NCH = BW // CB
    KD = D // _L
    W = 2 * D
    NR = 2 + NNEG

    mesh = plsc.VectorSubcoreMesh(
        core_axis_name="c", subcore_axis_name="s", num_cores=2, num_subcores=16
    )

    @functools.partial(
        pl.kernel,
        out_type=jax.ShapeDtypeStruct((B, 2 * _L), jnp.float32),
        mesh=mesh,
        scratch_types=[
            pltpu.VMEM((CB,), jnp.int32),
            pltpu.VMEM((CB,), jnp.int32),
            pltpu.VMEM((CB * NNEG,), jnp.int32),
            pltpu.VMEM((2, CB, W), jnp.float32),
            pltpu.VMEM((2, CB, W), jnp.float32),
            pltpu.VMEM((2, NNEG, CB, W), jnp.float32),
            pltpu.VMEM((2, NR, CB, _L), jnp.float32),
            pltpu.VMEM((CB, 2 * _L), jnp.float32),
            pltpu.SemaphoreType.DMA,
            pltpu.SemaphoreType.DMA,
        ],
        compiler_params=pltpu.CompilerParams(needs_layout_passes=False),
    )
    def sc_fn(u_pos_h, v_pos_h, vneg_h, u_tab, v_tab, out_h,
              u_idx, v_idx, n_idx, u_rows, v_rows, n_rows, par, pbuf,
              sem0, sem1):
        wid = lax.axis_index("c") * 16 + lax.axis_index("s")
        base = wid * BW

        sems = (sem0, sem1)

        def fire(c, p):
            s = sems[p]
            pltpu.sync_copy(u_pos_h.at[pl.ds(base + c * CB, CB)], u_idx)
            pltpu.sync_copy(v_pos_h.at[pl.ds(base + c * CB, CB)], v_idx)
            pltpu.sync_copy(
                vneg_h.at[pl.ds((base + c * CB) * NNEG, CB * NNEG)], n_idx)

            uvec = u_idx[...]
            vvec = v_idx[...]
            nvec = [n_idx[pl.ds(q * _L, _L)] for q in range(NNEG)]
            ur = _wide_row(uvec)
            vr = _wide_row(vvec)
            nr = [_wide_row(x) for x in nvec]
            uf = _half(uvec).astype(jnp.float32)
            vf = _half(vvec).astype(jnp.float32)
            nf = [_half(x).astype(jnp.float32) for x in nvec]
            for t in range(CB):
                pltpu.async_copy(u_tab.at[ur[t]], u_rows.at[p, t], s)
                pltpu.async_copy(v_tab.at[vr[t]], v_rows.at[p, t], s)
                par[p, 0, t] = jnp.broadcast_to(uf[t], (_L,))
                par[p, 1, t] = jnp.broadcast_to(vf[t], (_L,))
                for j in range(NNEG):
                    q, r = divmod(t * NNEG + j, _L)
                    pltpu.async_copy(
                        v_tab.at[nr[q][r]], n_rows.at[p, j, t], s)
                    par[p, 2 + j, t] = jnp.broadcast_to(nf[q][r], (_L,))

        def drain(p):
            s = sems[p]
            pltpu.make_async_copy(
                u_tab.at[pl.ds(0, CB)], u_rows.at[p], s).wait()
            pltpu.make_async_copy(
                u_tab.at[pl.ds(0, CB)], v_rows.at[p], s).wait()
            for j in range(NNEG):
                pltpu.make_async_copy(
                    u_tab.at[pl.ds(0, CB)], n_rows.at[p, j], s).wait()

        def compute_chunk(p):
            ub, vb, nb = u_rows.at[p], v_rows.at[p], n_rows.at[p]

            def body(i, carry):
                pu = par[p, 0, i]
                pv = par[p, 1, i]
                pn = [par[p, 2 + j, i] for j in range(NNEG)]
                pos = None
                neg = None
                for k in range(KD):
                    lo = pl.ds(k * _L, _L)
                    hi = pl.ds(D + k * _L, _L)
                    ul = ub[i, lo]
                    u = ul + (ub[i, hi] - ul) * pu
                    vl = vb[i, lo]
                    v = vl + (vb[i, hi] - vl) * pv
                    ns = None
                    for j in range(NNEG):
                        nl = nb[j, i, lo]
                        nv = nl + (nb[j, i, hi] - nl) * pn[j]
                        ns = nv if ns is None else ns + nv
                    pp = u * v
                    nn = u * ns
                    pos = pp if pos is None else pos + pp
                    neg = nn if neg is None else neg + nn
                pbuf[i, pl.ds(0, _L)] = pos
                pbuf[i, pl.ds(_L, _L)] = neg
                return carry

            lax.fori_loop(0, CB, body, 0)

        def emit(c, p):
            compute_chunk(p)
            pltpu.sync_copy(pbuf, out_h.at[pl.ds(base + c * CB, CB)])

        fire(0, 0)

        def pair(k, carry):
            c0 = 2 * k
            fire(c0 + 1, 1)
            drain(0)
            emit(c0, 0)

            @pl.when(k + 1 < NCH // 2)
            def _():
                fire(c0 + 2, 0)

            drain(1)
            emit(c0 + 1, 1)
            return carry

        lax.fori_loop(0, NCH // 2, pair, 0)

    return sc_fn


def _finish(part, B):
    def body(x_ref, o_ref):
        x = x_ref[...]
        pos = jnp.sum(x[:, :_L], axis=1)
        neg = jnp.sum(x[:, _L:], axis=1)
        tot = jax.nn.log_sigmoid(pos) + jax.nn.log_sigmoid(-neg)
        o_ref[0, 0] = -jnp.sum(tot) / B

    return pl.pallas_call(
        body,
        out_shape=jax.ShapeDtypeStruct((1, 1), jnp.float32),
        in_specs=[pl.BlockSpec(memory_space=pltpu.VMEM)],
        out_specs=pl.BlockSpec(memory_space=pltpu.SMEM),
    )(part)


def kernel(u_pos, v_pos, v_neg, batch_size, U, V):
    B = u_pos.shape[0]
    D = U.shape[1]
    NNEG = v_neg.shape[1]
    vneg_flat = v_neg.reshape(B * NNEG)
    u_packed = _pack_table(U.T)
    v_packed = _pack_table(V.T)
    sc_fn = _make_sc_gather_score(B, D, NNEG)
    part = sc_fn(u_pos, v_pos, vneg_flat, u_packed, v_packed)
    out = _finish(part, B)
    return out[0, 0]


# speedup vs baseline: 2.0162x; 1.1157x over previous
import functools

import jax
import jax.numpy as jnp
from jax import lax
from jax.experimental import pallas as pl
from jax.experimental.pallas import tpu as pltpu
from jax.experimental.pallas import tpu_sc as plsc

_L = 16


_BKV = 16384


def _pack_table(xt):
    D, V = xt.shape
    grid = -(-V // _BKV)

    def body(x_ref, o_ref):
        y = x_ref[...].T
        o_ref[...] = jnp.concatenate([y[:_BKV // 2], y[_BKV // 2:]], axis=1)

    return pl.pallas_call(
        body,
        grid=(grid,),
        in_specs=[pl.BlockSpec((D, _BKV), lambda i: (0, i))],
        out_specs=pl.BlockSpec((_BKV // 2, 2 * D), lambda i: (i, 0)),
        out_shape=jax.ShapeDtypeStruct((grid * _BKV // 2, 2 * D), jnp.float32),
    )(xt)


def _wide_row(x):
    return ((x >> 14) << 13) + (x & 8191)


def _half(x):
    return (x >> 13) & 1


def _make_sc_gather_score(B, D, NNEG):
    NW = 32
    BW = B // NW
    CB = 16
    NCH = BW // CB
    KD = D // _L
    W = 2 * D
    NR = 2 + NNEG

    mesh = plsc.VectorSubcoreMesh(
        core_axis_name="c", subcore_axis_name="s", num_cores=2, num_subcores=16
    )

    @functools.partial(
        pl.kernel,
        out_type=jax.ShapeDtypeStruct((B, 2 * _L), jnp.float32),
        mesh=mesh,
        scratch_types=[
            pltpu.VMEM((CB,), jnp.int32),
            pltpu.VMEM((CB,), jnp.int32),
            pltpu.VMEM((CB * NNEG,), jnp.int32),
            pltpu.VMEM((2, CB, W), jnp.float32),
            pltpu.VMEM((2, CB, W), jnp.float32),
            pltpu.VMEM((2, NNEG, CB, W), jnp.float32),
            pltpu.VMEM((2, NR, CB, _L), jnp.float32),
            pltpu.VMEM((CB, 2 * _L), jnp.float32),
            pltpu.SemaphoreType.DMA,
            pltpu.SemaphoreType.DMA,
        ],
        compiler_params=pltpu.CompilerParams(needs_layout_passes=False),
    )
    def sc_fn(u_pos_h, v_pos_h, vneg_h, u_tab, v_tab, out_h,
              u_idx, v_idx, n_idx, u_rows, v_rows, n_rows, par, pbuf,
              sem0, sem1):
        wid = lax.axis_index("c") * 16 + lax.axis_index("s")
        base = wid * BW

        sems = (sem0, sem1)

        def fire(c, p):
            s = sems[p]
            pltpu.sync_copy(u_pos_h.at[pl.ds(base + c * CB, CB)], u_idx)
            pltpu.sync_copy(v_pos_h.at[pl.ds(base + c * CB, CB)], v_idx)
            pltpu.sync_copy(
                vneg_h.at[pl.ds((base + c * CB) * NNEG, CB * NNEG)], n_idx)

            uvec = u_idx[...]
            vvec = v_idx[...]
            nvec = [n_idx[pl.ds(q * _L, _L)] for q in range(NNEG)]
            ur = _wide_row(uvec)
            vr = _wide_row(vvec)
            nr = [_wide_row(x) for x in nvec]
            uf = _half(uvec).astype(jnp.float32)
            vf = _half(vvec).astype(jnp.float32)
            nf = [_half(x).astype(jnp.float32) for x in nvec]
            for t in range(CB):
                pltpu.async_copy(u_tab.at[ur[t]], u_rows.at[p, t], s)
                pltpu.async_copy(v_tab.at[vr[t]], v_rows.at[p, t], s)
                par[p, 0, t] = jnp.broadcast_to(uf[t], (_L,))
                par[p, 1, t] = jnp.broadcast_to(vf[t], (_L,))
                for j in range(NNEG):
                    q, r = divmod(t * NNEG + j, _L)
                    pltpu.async_copy(
                        v_tab.at[nr[q][r]], n_rows.at[p, j, t], s)
                    par[p, 2 + j, t] = jnp.broadcast_to(nf[q][r], (_L,))

        def drain(p):
            s = sems[p]
            pltpu.make_async_copy(
                u_tab.at[pl.ds(0, CB)], u_rows.at[p], s).wait()
            pltpu.make_async_copy(
                u_tab.at[pl.ds(0, CB)], v_rows.at[p], s).wait()
            for j in range(NNEG):
                pltpu.make_async_copy(
                    u_tab.at[pl.ds(0, CB)], n_rows.at[p, j], s).wait()

        def compute_chunk(p):
            ub, vb, nb = u_rows.at[p], v_rows.at[p], n_rows.at[p]

            def body(i, carry):
                pu = par[p, 0, i]
                pv = par[p, 1, i]
                pn = [par[p, 2 + j, i] for j in range(NNEG)]
                pos = None
                neg = None
                for k in range(KD):
                    lo = pl.ds(k * _L, _L)
                    hi = pl.ds(D + k * _L, _L)
                    ul = ub[i, lo]
                    u = ul + (ub[i, hi] - ul) * pu
                    vl = vb[i, lo]
                    v = vl + (vb[i, hi] - vl) * pv
                    ns = None
                    for j in range(NNEG):
                        nl = nb[j, i, lo]
                        nv = nl + (nb[j, i, hi] - nl) * pn[j]
                        ns = nv if ns is None else ns + nv
                    pp = u * v
                    nn = u * ns
                    pos = pp if pos is None else pos + pp
                    neg = nn if neg is None else neg + nn
                pbuf[i, pl.ds(0, _L)] = pos
                pbuf[i, pl.ds(_L, _L)] = neg
                return carry

            lax.fori_loop(0, CB, body, 0)

        def emit(c, p):
            compute_chunk(p)
            pltpu.sync_copy(pbuf, out_h.at[pl.ds(base + c * CB, CB)])

        fire(0, 0)

        def pair(k, carry):
            c0 = 2 * k
            fire(c0 + 1, 1)
            drain(0)
            emit(c0, 0)

            @pl.when(k + 1 < NCH // 2)
            def _():
                fire(c0 + 2, 0)

            drain(1)
            emit(c0 + 1, 1)
            return carry

        lax.fori_loop(0, NCH // 2, pair, 0)

    return sc_fn


def _finish(part, B):
    def body(x_ref, o_ref):
        x = x_ref[...]
        pos = jnp.sum(x[:, :_L], axis=1)
        neg = jnp.sum(x[:, _L:], axis=1)
        tot = jax.nn.log_sigmoid(pos) + jax.nn.log_sigmoid(-neg)
        o_ref[0, 0] = -jnp.sum(tot) / B

    return pl.pallas_call(
        body,
        out_shape=jax.ShapeDtypeStruct((1, 1), jnp.float32),
        in_specs=[pl.BlockSpec(memory_space=pltpu.VMEM)],
        out_specs=pl.BlockSpec(memory_space=pltpu.SMEM),
    )(part)


def kernel(u_pos, v_pos, v_neg, batch_size, U, V):
    B = u_pos.shape[0]
    D = U.shape[1]
    NNEG = v_neg.shape[1]
    vneg_flat = v_neg.reshape(B * NNEG)
    u_packed = _pack_table(U.T)
    v_packed = _pack_table(V.T)
    sc_fn = _make_sc_gather_score(B, D, NNEG)
    part = sc_fn(u_pos, v_pos, vneg_flat, u_packed, v_packed)
    out = _finish(part, B)
    return out[0, 0]


# speedup vs baseline: 2.1252x; 1.0541x over previous
import functools

import jax
import jax.numpy as jnp
from jax import lax
from jax.experimental import pallas as pl
from jax.experimental.pallas import tpu as pltpu
from jax.experimental.pallas import tpu_sc as plsc

_L = 16


_BKV = 32768


def _pack_table(xt):
    D, V = xt.shape
    grid = -(-V // _BKV)

    def body(x_ref, o_ref):
        y = x_ref[...].T
        o_ref[...] = jnp.concatenate([y[:_BKV // 2], y[_BKV // 2:]], axis=1)

    return pl.pallas_call(
        body,
        grid=(grid,),
        in_specs=[pl.BlockSpec((D, _BKV), lambda i: (0, i))],
        out_specs=pl.BlockSpec((_BKV // 2, 2 * D), lambda i: (i, 0)),
        out_shape=jax.ShapeDtypeStruct((grid * _BKV // 2, 2 * D), jnp.float32),
    )(xt)


def _wide_row(x):
    return ((x >> 15) << 14) + (x & 16383)


def _half(x):
    return (x >> 14) & 1


def _make_sc_gather_score(B, D, NNEG):
    NW = 32
    BW = B // NW
    CB = 16
    NCH = BW // CB
    KD = D // _L
    W = 2 * D
    NR = 2 + NNEG

    mesh = plsc.VectorSubcoreMesh(
        core_axis_name="c", subcore_axis_name="s", num_cores=2, num_subcores=16
    )

    @functools.partial(
        pl.kernel,
        out_type=jax.ShapeDtypeStruct((B, 2 * _L), jnp.float32),
        mesh=mesh,
        scratch_types=[
            pltpu.VMEM((CB,), jnp.int32),
            pltpu.VMEM((CB,), jnp.int32),
            pltpu.VMEM((CB * NNEG,), jnp.int32),
            pltpu.VMEM((2, CB, W), jnp.float32),
            pltpu.VMEM((2, CB, W), jnp.float32),
            pltpu.VMEM((2, NNEG, CB, W), jnp.float32),
            pltpu.VMEM((2, NR, CB, _L), jnp.float32),
            pltpu.VMEM((CB, 2 * _L), jnp.float32),
            pltpu.SemaphoreType.DMA,
            pltpu.SemaphoreType.DMA,
        ],
        compiler_params=pltpu.CompilerParams(needs_layout_passes=False),
    )
    def sc_fn(u_pos_h, v_pos_h, vneg_h, u_tab, v_tab, out_h,
              u_idx, v_idx, n_idx, u_rows, v_rows, n_rows, par, pbuf,
              sem0, sem1):
        wid = lax.axis_index("c") * 16 + lax.axis_index("s")
        base = wid * BW

        sems = (sem0, sem1)

        def fire(c, p):
            s = sems[p]
            pltpu.sync_copy(u_pos_h.at[pl.ds(base + c * CB, CB)], u_idx)
            pltpu.sync_copy(v_pos_h.at[pl.ds(base + c * CB, CB)], v_idx)
            pltpu.sync_copy(
                vneg_h.at[pl.ds((base + c * CB) * NNEG, CB * NNEG)], n_idx)

            uvec = u_idx[...]
            vvec = v_idx[...]
            nvec = [n_idx[pl.ds(q * _L, _L)] for q in range(NNEG)]
            ur = _wide_row(uvec)
            vr = _wide_row(vvec)
            nr = [_wide_row(x) for x in nvec]
            uf = _half(uvec).astype(jnp.float32)
            vf = _half(vvec).astype(jnp.float32)
            nf = [_half(x).astype(jnp.float32) for x in nvec]
            for t in range(CB):
                pltpu.async_copy(u_tab.at[ur[t]], u_rows.at[p, t], s)
                pltpu.async_copy(v_tab.at[vr[t]], v_rows.at[p, t], s)
                par[p, 0, t] = jnp.broadcast_to(uf[t], (_L,))
                par[p, 1, t] = jnp.broadcast_to(vf[t], (_L,))
                for j in range(NNEG):
                    q, r = divmod(t * NNEG + j, _L)
                    pltpu.async_copy(
                        v_tab.at[nr[q][r]], n_rows.at[p, j, t], s)
                    par[p, 2 + j, t] = jnp.broadcast_to(nf[q][r], (_L,))

        def drain(p):
            s = sems[p]
            pltpu.make_async_copy(
                u_tab.at[pl.ds(0, CB)], u_rows.at[p], s).wait()
            pltpu.make_async_copy(
                u_tab.at[pl.ds(0, CB)], v_rows.at[p], s).wait()
            for j in range(NNEG):
                pltpu.make_async_copy(
                    u_tab.at[pl.ds(0, CB)], n_rows.at[p, j], s).wait()

        def compute_chunk(p):
            ub, vb, nb = u_rows.at[p], v_rows.at[p], n_rows.at[p]

            def body(i, carry):
                pu = par[p, 0, i]
                pv = par[p, 1, i]
                pn = [par[p, 2 + j, i] for j in range(NNEG)]
                pos = None
                neg = None
                for k in range(KD):
                    lo = pl.ds(k * _L, _L)
                    hi = pl.ds(D + k * _L, _L)
                    ul = ub[i, lo]
                    u = ul + (ub[i, hi] - ul) * pu
                    vl = vb[i, lo]
                    v = vl + (vb[i, hi] - vl) * pv
                    ns = None
                    for j in range(NNEG):
                        nl = nb[j, i, lo]
                        nv = nl + (nb[j, i, hi] - nl) * pn[j]
                        ns = nv if ns is None else ns + nv
                    pp = u * v
                    nn = u * ns
                    pos = pp if pos is None else pos + pp
                    neg = nn if neg is None else neg + nn
                pbuf[i, pl.ds(0, _L)] = pos
                pbuf[i, pl.ds(_L, _L)] = neg
                return carry

            lax.fori_loop(0, CB, body, 0)

        def emit(c, p):
            compute_chunk(p)
            pltpu.sync_copy(pbuf, out_h.at[pl.ds(base + c * CB, CB)])

        fire(0, 0)

        def pair(k, carry):
            c0 = 2 * k
            fire(c0 + 1, 1)
            drain(0)
            emit(c0, 0)

            @pl.when(k + 1 < NCH // 2)
            def _():
                fire(c0 + 2, 0)

            drain(1)
            emit(c0 + 1, 1)
            return carry

        lax.fori_loop(0, NCH // 2, pair, 0)

    return sc_fn


def _finish(part, B):
    def body(x_ref, o_ref):
        x = x_ref[...]
        pos = jnp.sum(x[:, :_L], axis=1)
        neg = jnp.sum(x[:, _L:], axis=1)
        tot = jax.nn.log_sigmoid(pos) + jax.nn.log_sigmoid(-neg)
        o_ref[0, 0] = -jnp.sum(tot) / B

    return pl.pallas_call(
        body,
        out_shape=jax.ShapeDtypeStruct((1, 1), jnp.float32),
        in_specs=[pl.BlockSpec(memory_space=pltpu.VMEM)],
        out_specs=pl.BlockSpec(memory_space=pltpu.SMEM),
    )(part)


def kernel(u_pos, v_pos, v_neg, batch_size, U, V):
    B = u_pos.shape[0]
    D = U.shape[1]
    NNEG = v_neg.shape[1]
    vneg_flat = v_neg.reshape(B * NNEG)
    u_packed = _pack_table(U.T)
    v_packed = _pack_table(V.T)
    sc_fn = _make_sc_gather_score(B, D, NNEG)
    part = sc_fn(u_pos, v_pos, vneg_flat, u_packed, v_packed)
    out = _finish(part, B)
    return out[0, 0]
